# Initial kernel scaffold; baseline (speedup 1.0000x reference)
import jax
import jax.numpy as jnp
from jax.experimental import pallas as pl


def kernel(center_word, context_word, negative_samples, embeddings, context_embeddings):
    raise NotImplementedError("write your pallas kernel here")


# speedup vs baseline: 1.2970x; 1.2970x over previous
import jax
import jax.numpy as jnp
from jax import lax
from jax.experimental import pallas as pl
from jax.experimental.pallas import tpu as pltpu
from jax.experimental.pallas import tpu_sc as plsc

VOCAB = 100000
D = 128
B = 4096
NS = 5
NC = 2
NSUB = 16
NW = NC * NSUB
BPW = B // NW
L = 16
NG = BPW // L


def _sg_body(cidx_hbm, pidx_hbm, nidx_hbm, emb_hbm, ctx_hbm,
             pos_out, negt_out,
             cidx_v, pidx_v, nidx_v, crows, prows, nrows,
             pscore_v, nscore_v, sem):
    wid = lax.axis_index("s") * NC + lax.axis_index("c")
    base = wid * BPW

    pltpu.sync_copy(cidx_hbm.at[pl.ds(base, BPW)], cidx_v)
    pltpu.sync_copy(pidx_hbm.at[pl.ds(base, BPW)], pidx_v)
    for j in range(NS):
        pltpu.sync_copy(nidx_hbm.at[pl.ds(j * B + base, BPW)], nidx_v.at[j])

    cp_c = pltpu.async_copy(emb_hbm.at[cidx_v], crows, sem)
    cp_p = pltpu.async_copy(ctx_hbm.at[pidx_v], prows, sem)
    cps = [pltpu.async_copy(ctx_hbm.at[nidx_v.at[j]], nrows.at[j], sem)
           for j in range(NS)]
    cp_c.wait()
    cp_p.wait()
    for cp in cps:
        cp.wait()

    lane = lax.iota(jnp.int32, L)
    zero = jnp.zeros((L,), jnp.float32)
    for g in range(NG):
        row_idx = lane + (g * L)

        def body(d, accs):
            acc_p, acc_n = accs
            col = jnp.full((L,), d, jnp.int32)
            c = plsc.load_gather(crows, [row_idx, col])
            acc_p = acc_p + c * plsc.load_gather(prows, [row_idx, col])
            acc_n = tuple(
                acc_n[j] + c * plsc.load_gather(nrows.at[j], [row_idx, col])
                for j in range(NS))
            return (acc_p, acc_n)

        acc_p, acc_n = lax.fori_loop(
            0, D, body, (zero, (zero,) * NS), unroll=False)
        pscore_v[pl.ds(g * L, L)] = acc_p
        for j in range(NS):
            nscore_v[j, pl.ds(g * L, L)] = acc_n[j]

    pltpu.sync_copy(pscore_v, pos_out.at[pl.ds(base, BPW)])
    for j in range(NS):
        pltpu.sync_copy(nscore_v.at[j], negt_out.at[pl.ds(j * B + base, BPW)])


@jax.jit
def _skipgram(center_word, context_word, neg_flat, embeddings, context_embeddings):
    mesh = plsc.VectorSubcoreMesh(
        core_axis_name="c", subcore_axis_name="s",
        num_cores=NC, num_subcores=NSUB)
    return pl.kernel(
        _sg_body,
        out_type=(
            jax.ShapeDtypeStruct((B,), jnp.float32),
            jax.ShapeDtypeStruct((NS * B,), jnp.float32),
        ),
        mesh=mesh,
        compiler_params=pltpu.CompilerParams(needs_layout_passes=False),
        scratch_types=[
            pltpu.VMEM((BPW,), jnp.int32),
            pltpu.VMEM((BPW,), jnp.int32),
            pltpu.VMEM((NS, BPW), jnp.int32),
            pltpu.VMEM((BPW, D), jnp.float32),
            pltpu.VMEM((BPW, D), jnp.float32),
            pltpu.VMEM((NS, BPW, D), jnp.float32),
            pltpu.VMEM((BPW,), jnp.float32),
            pltpu.VMEM((NS, BPW), jnp.float32),
            pltpu.SemaphoreType.DMA,
        ],
    )(center_word, context_word, neg_flat, embeddings, context_embeddings)


def kernel(center_word, context_word, negative_samples, embeddings, context_embeddings):
    neg_flat = negative_samples.T.reshape(-1)
    pos_score, negt = _skipgram(center_word, context_word, neg_flat,
                                embeddings, context_embeddings)
    neg_score = negt.reshape(NS, B).T
    return (pos_score, neg_score)


# speedup vs baseline: 1.4224x; 1.0966x over previous
import jax
import jax.numpy as jnp
from jax import lax
from jax.experimental import pallas as pl
from jax.experimental.pallas import tpu as pltpu
from jax.experimental.pallas import tpu_sc as plsc

VOCAB = 100000
D = 128
B = 4096
NS = 5
NC = 2
NSUB = 16
NW = NC * NSUB
BPW = B // NW
L = 16
NG = BPW // L


def _sg_body(cidx_hbm, pidx_hbm, nidx_hbm, emb_hbm, ctx_hbm,
             pos_out, negt_out,
             cidx_v, pidx_v, nidx_v, crows, prows, nrows,
             pscore_v, nscore_v, sem):
    wid = lax.axis_index("s") * NC + lax.axis_index("c")
    base = wid * BPW

    pltpu.sync_copy(cidx_hbm.at[pl.ds(base, BPW)], cidx_v)
    pltpu.sync_copy(pidx_hbm.at[pl.ds(base, BPW)], pidx_v)
    for j in range(NS):
        pltpu.sync_copy(nidx_hbm.at[pl.ds(j * B + base, BPW)], nidx_v.at[j])

    cp_c = pltpu.async_copy(emb_hbm.at[cidx_v], crows, sem)
    cp_p = pltpu.async_copy(ctx_hbm.at[pidx_v], prows, sem)
    cps = [pltpu.async_copy(ctx_hbm.at[nidx_v.at[j]], nrows.at[j], sem)
           for j in range(NS)]
    cp_c.wait()
    cp_p.wait()
    for cp in cps:
        cp.wait()

    lane = lax.iota(jnp.int32, L)
    zero = jnp.zeros((L,), jnp.float32)
    for g in range(NG):
        row_idx = lane + (g * L)

        def body(d, accs):
            acc_p, acc_n = accs
            col = jnp.full((L,), d, jnp.int32)
            c = plsc.load_gather(crows, [row_idx, col])
            acc_p = acc_p + c * plsc.load_gather(prows, [row_idx, col])
            acc_n = tuple(
                acc_n[j] + c * plsc.load_gather(nrows.at[j], [row_idx, col])
                for j in range(NS))
            return (acc_p, acc_n)

        acc_p, acc_n = lax.fori_loop(
            0, D, body, (zero, (zero,) * NS), unroll=8)
        pscore_v[pl.ds(g * L, L)] = acc_p
        for j in range(NS):
            nscore_v[j, pl.ds(g * L, L)] = acc_n[j]

    pltpu.sync_copy(pscore_v, pos_out.at[pl.ds(base, BPW)])
    for j in range(NS):
        pltpu.sync_copy(nscore_v.at[j], negt_out.at[pl.ds(j * B + base, BPW)])


@jax.jit
def _skipgram(center_word, context_word, neg_flat, embeddings, context_embeddings):
    mesh = plsc.VectorSubcoreMesh(
        core_axis_name="c", subcore_axis_name="s",
        num_cores=NC, num_subcores=NSUB)
    return pl.kernel(
        _sg_body,
        out_type=(
            jax.ShapeDtypeStruct((B,), jnp.float32),
            jax.ShapeDtypeStruct((NS * B,), jnp.float32),
        ),
        mesh=mesh,
        compiler_params=pltpu.CompilerParams(needs_layout_passes=False),
        scratch_types=[
            pltpu.VMEM((BPW,), jnp.int32),
            pltpu.VMEM((BPW,), jnp.int32),
            pltpu.VMEM((NS, BPW), jnp.int32),
            pltpu.VMEM((BPW, D), jnp.float32),
            pltpu.VMEM((BPW, D), jnp.float32),
            pltpu.VMEM((NS, BPW, D), jnp.float32),
            pltpu.VMEM((BPW,), jnp.float32),
            pltpu.VMEM((NS, BPW), jnp.float32),
            pltpu.SemaphoreType.DMA,
        ],
    )(center_word, context_word, neg_flat, embeddings, context_embeddings)


def kernel(center_word, context_word, negative_samples, embeddings, context_embeddings):
    neg_flat = negative_samples.T.reshape(-1)
    pos_score, negt = _skipgram(center_word, context_word, neg_flat,
                                embeddings, context_embeddings)
    neg_score = negt.reshape(NS, B).T
    return (pos_score, neg_score)


# speedup vs baseline: 2.7667x; 1.9452x over previous
import jax
import jax.numpy as jnp
from jax import lax
from jax.experimental import pallas as pl
from jax.experimental.pallas import tpu as pltpu
from jax.experimental.pallas import tpu_sc as plsc

VOCAB = 100000
D = 128
B = 4096
NS = 5
NSC = 6
NIDX = 7
NC = 2
NSUB = 16
NW = NC * NSUB
BPW = B // NW
L = 16
KCH = D // L
NG = BPW // L
PPITCH = BPW + 1


def _sg_body(idx_hbm, emb_hbm, ctx_hbm, out_hbm,
             idx_v, crows, prows, nrows, part, sall,
             sem_i, sem_cp, sem_n):
    wid = lax.axis_index("s") * NC + lax.axis_index("c")

    pltpu.async_copy(idx_hbm.at[wid], idx_v, sem_i).wait()
    cp_c = pltpu.async_copy(emb_hbm.at[idx_v.at[0]], crows, sem_cp)
    cp_p = pltpu.async_copy(ctx_hbm.at[idx_v.at[1]], prows, sem_cp)
    cps_n = [pltpu.async_copy(ctx_hbm.at[idx_v.at[2 + j]], nrows.at[j], sem_n)
             for j in range(NS)]

    lane = lax.iota(jnp.int32, L)

    def scatter_part(s, b, acc):
        idx = lane * PPITCH + (b + (s * L * PPITCH))
        plsc.store_scatter(part, [idx], acc)

    def run_group(srcs_refs, s0):
        def body(b, carry):
            c = [crows[b, pl.ds(k * L, L)] for k in range(KCH)]
            for i, r in enumerate(srcs_refs):
                acc = c[0] * r[b, pl.ds(0, L)]
                for k in range(1, KCH):
                    acc = acc + c[k] * r[b, pl.ds(k * L, L)]
                scatter_part(i, b, acc)
            return carry

        lax.fori_loop(0, BPW, body, 0, unroll=2)
        for i in range(len(srcs_refs)):
            for g in range(NG):
                acc = part[pl.ds(i * L * PPITCH + g * L, L)]
                for t in range(1, L):
                    acc = acc + part[pl.ds((i * L + t) * PPITCH + g * L, L)]
                sall[s0 + i, pl.ds(g * L, L)] = acc

    cp_c.wait()
    cp_p.wait()
    cps_n[0].wait()
    cps_n[1].wait()
    run_group([prows, nrows.at[0], nrows.at[1]], 0)
    cps_n[2].wait()
    cps_n[3].wait()
    cps_n[4].wait()
    run_group([nrows.at[2], nrows.at[3], nrows.at[4]], 3)

    pltpu.sync_copy(sall, out_hbm.at[wid])


@jax.jit
def _skipgram(idx_all, embeddings, context_embeddings):
    mesh = plsc.VectorSubcoreMesh(
        core_axis_name="c", subcore_axis_name="s",
        num_cores=NC, num_subcores=NSUB)
    return pl.kernel(
        _sg_body,
        out_type=jax.ShapeDtypeStruct((NW, NSC, BPW), jnp.float32),
        mesh=mesh,
        compiler_params=pltpu.CompilerParams(needs_layout_passes=False),
        scratch_types=[
            pltpu.VMEM((NIDX, BPW), jnp.int32),
            pltpu.VMEM((BPW, D), jnp.float32),
            pltpu.VMEM((BPW, D), jnp.float32),
            pltpu.VMEM((NS, BPW, D), jnp.float32),
            pltpu.VMEM((3 * L * PPITCH,), jnp.float32),
            pltpu.VMEM((NSC, BPW), jnp.float32),
            pltpu.SemaphoreType.DMA,
            pltpu.SemaphoreType.DMA,
            pltpu.SemaphoreType.DMA,
        ],
    )(idx_all, embeddings, context_embeddings)


def kernel(center_word, context_word, negative_samples, embeddings, context_embeddings):
    idx_all = jnp.concatenate(
        [center_word[None, :], context_word[None, :], negative_samples.T],
        axis=0)
    idx_all = idx_all.reshape(NIDX, NW, BPW).transpose(1, 0, 2)
    out = _skipgram(idx_all, embeddings, context_embeddings)
    pos_score = out[:, 0, :].reshape(B)
    neg_score = out[:, 1:, :].transpose(0, 2, 1).reshape(B, NS)
    return (pos_score, neg_score)
